# baseline (speedup 1.0000x reference)
import functools

import jax
import jax.numpy as jnp
from jax import lax
from jax.experimental import pallas as pl
from jax.experimental.pallas import tpu as pltpu
from jax.experimental.pallas import tpu_sc as plsc

NC, NS = 2, 16
NW = NC * NS
D = 128
CHUNK = 128
ZROWS = 128


def _make_seg_sum(n_edges_pad, n_dst_pad):
    epw = n_edges_pad // NW
    n_chunks = epw // CHUNK
    assert n_chunks % 2 == 0 and n_chunks >= 4
    rpt = n_dst_pad // NS
    nz = rpt // ZROWS

    mesh = plsc.VectorSubcoreMesh(
        core_axis_name="c", subcore_axis_name="s",
        num_cores=NC, num_subcores=NS)

    @functools.partial(
        pl.kernel,
        out_type=(
            jax.ShapeDtypeStruct((NC * n_dst_pad, D), jnp.float32),
            jax.ShapeDtypeStruct((NW, n_dst_pad), jnp.float32),
        ),
        mesh=mesh,
        compiler_params=pltpu.CompilerParams(needs_layout_passes=False),
        scratch_types=[
            pltpu.VMEM((ZROWS, D), jnp.float32),
            pltpu.VMEM((n_dst_pad,), jnp.float32),
            pltpu.VMEM((CHUNK,), jnp.int32),
            pltpu.VMEM((CHUNK,), jnp.int32),
            pltpu.VMEM((CHUNK,), jnp.int32),
            pltpu.VMEM((CHUNK,), jnp.int32),
            pltpu.VMEM((CHUNK, D), jnp.float32),
            pltpu.VMEM_SHARED((n_dst_pad, D), jnp.float32),
            pltpu.SemaphoreType.DMA,
            pltpu.SemaphoreType.DMA,
        ],
    )
    def seg_sum(x_hbm, src_hbm, dst_hbm, z_hbm, zcnt_hbm,
                sum_out, cnt_out, zf_v, cnt_v,
                is0, is1, id0, id1, rows_v, sh_sum, sem_g, sem_i):
        is_v = (is0, is1)
        id_v = (id0, id1)

        cid = lax.axis_index("c")
        sid = lax.axis_index("s")
        wid = sid * NC + cid

        pltpu.sync_copy(z_hbm, zf_v)
        pltpu.sync_copy(zcnt_hbm, cnt_v)
        for k in range(nz):
            pltpu.sync_copy(zf_v, sh_sum.at[pl.ds(sid * rpt + k * ZROWS,
                                                  ZROWS)])
        plsc.subcore_barrier()

        base = wid * epw
        ones16 = jnp.ones((16,), jnp.float32)

        def count(ib):
            for j in range(CHUNK // 16):
                dvec = id_v[ib][pl.ds(j * 16, 16)]
                plsc.addupdate_scatter(cnt_v, [dvec], ones16)

        def body(j, carry):
            off = base + j * CHUNK
            pltpu.sync_copy(src_hbm.at[pl.ds(off, CHUNK)], is0)
            pltpu.sync_copy(dst_hbm.at[pl.ds(off, CHUNK)], id0)
            pltpu.async_copy(x_hbm.at[is0], rows_v, sem_g).wait()
            pltpu.sync_copy(rows_v, sh_sum.at[id0], add=True)
            count(0)
            return carry

        lax.fori_loop(0, n_chunks, body, 0)

        plsc.subcore_barrier()

        obase = cid * n_dst_pad + sid * rpt
        pltpu.sync_copy(sh_sum.at[pl.ds(sid * rpt, rpt)],
                        sum_out.at[pl.ds(obase, rpt)])
        pltpu.sync_copy(cnt_v, cnt_out.at[wid])

    return seg_sum


def _tc_body(act, sum0, sum1, cnt, xr, wl, wr, br, o):
    s = sum0[...] + sum1[...]
    c = jnp.maximum(jnp.sum(cnt[...], axis=0), 1.0)
    agg = s / c[:, None]
    y = (lax.dot_general(agg, wl[...], (((1,), (1,)), ((), ())),
                         preferred_element_type=jnp.float32)
         + lax.dot_general(xr[...], wr[...], (((1,), (1,)), ((), ())),
                           preferred_element_type=jnp.float32)
         + br[...])
    o[...] = act(y)


def _make_dense(n_dst_pad, act):
    B = 1024
    grid = n_dst_pad // B
    nblk = grid

    def call(sum_flat, cnt_parts, x_dst, wl, wr, b2d):
        return pl.pallas_call(
            functools.partial(_tc_body, act),
            grid=(grid,),
            in_specs=[
                pl.BlockSpec((B, D), lambda i: (i, 0)),
                pl.BlockSpec((B, D), lambda i: (i + nblk, 0)),
                pl.BlockSpec((NW, B), lambda i: (0, i)),
                pl.BlockSpec((B, D), lambda i: (i, 0)),
                pl.BlockSpec((D, D), lambda i: (0, 0)),
                pl.BlockSpec((D, D), lambda i: (0, 0)),
                pl.BlockSpec((1, D), lambda i: (0, 0)),
            ],
            out_specs=pl.BlockSpec((B, D), lambda i: (i, 0)),
            out_shape=jax.ShapeDtypeStruct((n_dst_pad, D), jnp.float32),
        )(sum_flat, sum_flat, cnt_parts, x_dst, wl, wr, b2d)

    return call


N0, N1, N2 = 50000, 10000, 2000
E1, E2 = 320000, 64000
P1, P2 = 10240, 2048
E1P = 80 * CHUNK * NW
E2P = 16 * CHUNK * NW

_seg1 = _make_seg_sum(E1P, P1)
_seg2 = _make_seg_sum(E2P, P2)
_dense1 = _make_dense(P1, jax.nn.relu)
_dense2 = _make_dense(P2, jax.nn.sigmoid)


def _pad_edges(ei, e_pad, n_dst, n_dst_pad):
    npad = e_pad - ei.shape[1]
    spare = n_dst_pad - n_dst
    pad_dst = n_dst + (jnp.arange(npad, dtype=jnp.int32) % spare)
    pad = jnp.stack([jnp.zeros((npad,), jnp.int32), pad_dst])
    return jnp.concatenate([ei, pad], axis=1)


def kernel(x, edge_index1, edge_index2, W1l, W1r, b1, W2l, W2r, b2):
    z = jnp.zeros((ZROWS, D), jnp.float32)
    zc1 = jnp.zeros((P1,), jnp.float32)
    zc2 = jnp.zeros((P2,), jnp.float32)
    ei1 = _pad_edges(edge_index1, E1P, N1, P1)
    ei2 = _pad_edges(edge_index2, E2P, N2, P2)

    sum1, cnt1 = _seg1(x, ei1[0], ei1[1], z, zc1)
    h = _dense1(sum1, cnt1, x, W1l, W1r, b1.reshape(1, D))
    sum2, cnt2 = _seg2(h, ei2[0], ei2[1], z, zc2)
    out = _dense2(sum2, cnt2, h, W2l, W2r, b2.reshape(1, D))
    return out[:N2]


# speedup vs baseline: 1.9189x; 1.9189x over previous
import functools

import jax
import jax.numpy as jnp
from jax import lax
from jax.experimental import pallas as pl
from jax.experimental.pallas import tpu as pltpu
from jax.experimental.pallas import tpu_sc as plsc

NC, NS = 2, 16
NW = NC * NS
D = 128
CHUNK = 128
ZROWS = 128


def _make_seg_sum(n_src, n_edges, n_dst_pad):
    epw = n_edges // NW
    n_full = epw // CHUNK
    tail = epw - n_full * CHUNK
    rpt = n_dst_pad // NS
    nz = rpt // ZROWS

    mesh = plsc.VectorSubcoreMesh(
        core_axis_name="c", subcore_axis_name="s",
        num_cores=NC, num_subcores=NS)

    @functools.partial(
        pl.kernel,
        out_type=(
            jax.ShapeDtypeStruct((NC * n_dst_pad, D), jnp.float32),
            jax.ShapeDtypeStruct((NW, n_dst_pad), jnp.float32),
        ),
        mesh=mesh,
        compiler_params=pltpu.CompilerParams(needs_layout_passes=False),
        scratch_types=[
            pltpu.VMEM((ZROWS, D), jnp.float32),
            pltpu.VMEM((n_dst_pad,), jnp.float32),
            pltpu.VMEM((CHUNK,), jnp.int32),
            pltpu.VMEM((CHUNK,), jnp.int32),
            pltpu.VMEM((tail,), jnp.int32),
            pltpu.VMEM((tail,), jnp.int32),
            pltpu.VMEM((CHUNK, D), jnp.float32),
            pltpu.VMEM((tail, D), jnp.float32),
            pltpu.VMEM_SHARED((n_dst_pad, D), jnp.float32),
            pltpu.SemaphoreType.DMA,
        ],
    )
    def seg_sum(x_hbm, src_hbm, dst_hbm, z128_hbm, zcnt_hbm,
                sum_out, cnt_out,
                zf_v, cnt_v, is_v, id_v, is_t, id_t,
                rows_v, rows_t, sh_sum, sem):
        cid = lax.axis_index("c")
        sid = lax.axis_index("s")
        wid = sid * NC + cid

        pltpu.sync_copy(z128_hbm, zf_v)
        pltpu.sync_copy(zcnt_hbm, cnt_v)
        for k in range(nz):
            off = sid * rpt + k * ZROWS
            pltpu.sync_copy(zf_v, sh_sum.at[pl.ds(off, ZROWS)])
        plsc.subcore_barrier()

        base = wid * epw
        ones16 = jnp.ones((16,), jnp.float32)

        def count(idx_ref, m):
            for j in range(m // 16):
                dvec = idx_ref[pl.ds(j * 16, 16)]
                plsc.addupdate_scatter(cnt_v, [dvec], ones16)

        def body(i, carry):
            off = base + i * CHUNK
            pltpu.sync_copy(src_hbm.at[pl.ds(off, CHUNK)], is_v)
            pltpu.sync_copy(dst_hbm.at[pl.ds(off, CHUNK)], id_v)
            pltpu.async_copy(x_hbm.at[is_v], rows_v, sem).wait()
            pltpu.sync_copy(rows_v, sh_sum.at[id_v], add=True)
            count(id_v, CHUNK)
            return carry

        lax.fori_loop(0, n_full, body, 0)

        toff = base + n_full * CHUNK
        pltpu.sync_copy(src_hbm.at[pl.ds(toff, tail)], is_t)
        pltpu.sync_copy(dst_hbm.at[pl.ds(toff, tail)], id_t)
        pltpu.async_copy(x_hbm.at[is_t], rows_t, sem).wait()
        pltpu.sync_copy(rows_t, sh_sum.at[id_t], add=True)
        count(id_t, tail)

        plsc.subcore_barrier()

        obase = cid * n_dst_pad + sid * rpt
        pltpu.sync_copy(sh_sum.at[pl.ds(sid * rpt, rpt)],
                        sum_out.at[pl.ds(obase, rpt)])
        pltpu.sync_copy(cnt_v, cnt_out.at[wid])

    return seg_sum


def _tc_body(act, sum0, sum1, cnt, xr, wl, wr, br, o):
    s = sum0[...] + sum1[...]
    c = jnp.maximum(jnp.sum(cnt[...], axis=0), 1.0)
    agg = s / c[:, None]
    y = (lax.dot_general(agg, wl[...], (((1,), (1,)), ((), ())),
                         preferred_element_type=jnp.float32)
         + lax.dot_general(xr[...], wr[...], (((1,), (1,)), ((), ())),
                           preferred_element_type=jnp.float32)
         + br[...])
    o[...] = act(y)


def _make_dense(n_dst_pad, act):
    B = 1024
    grid = n_dst_pad // B
    nblk = grid

    def call(sum_flat, cnt_parts, x_dst, wl, wr, b2d):
        return pl.pallas_call(
            functools.partial(_tc_body, act),
            grid=(grid,),
            in_specs=[
                pl.BlockSpec((B, D), lambda i: (i, 0)),
                pl.BlockSpec((B, D), lambda i: (i + nblk, 0)),
                pl.BlockSpec((NW, B), lambda i: (0, i)),
                pl.BlockSpec((B, D), lambda i: (i, 0)),
                pl.BlockSpec((D, D), lambda i: (0, 0)),
                pl.BlockSpec((D, D), lambda i: (0, 0)),
                pl.BlockSpec((1, D), lambda i: (0, 0)),
            ],
            out_specs=pl.BlockSpec((B, D), lambda i: (i, 0)),
            out_shape=jax.ShapeDtypeStruct((n_dst_pad, D), jnp.float32),
        )(sum_flat, sum_flat, cnt_parts, x_dst, wl, wr, b2d)

    return call


N0, N1, N2 = 50000, 10000, 2000
E1, E2 = 320000, 64000
P1, P2 = 10240, 2048

_seg1 = _make_seg_sum(N0, E1, P1)
_seg2 = _make_seg_sum(P1, E2, P2)
_dense1 = _make_dense(P1, jax.nn.relu)
_dense2 = _make_dense(P2, jax.nn.sigmoid)


def kernel(x, edge_index1, edge_index2, W1l, W1r, b1, W2l, W2r, b2):
    z128 = jnp.zeros((ZROWS, D), jnp.float32)
    zc1 = jnp.zeros((P1,), jnp.float32)
    zc2 = jnp.zeros((P2,), jnp.float32)

    sum1, cnt1 = _seg1(x, edge_index1[0], edge_index1[1], z128, zc1)
    h = _dense1(sum1, cnt1, x, W1l, W1r, b1.reshape(1, D))
    sum2, cnt2 = _seg2(h, edge_index2[0], edge_index2[1], z128, zc2)
    out = _dense2(sum2, cnt2, h, W2l, W2r, b2.reshape(1, D))
    return out[:N2]
